# baseline (speedup 1.0000x reference)
import jax
import jax.numpy as jnp
from jax.experimental import pallas as pl
from jax.experimental.pallas import tpu as pltpu

_N, _P, _B, _D, _M = 100000, 256, 4096, 128, 32
_BN = 14336
_G = -(-_N // _BN)


def _path_body(meta_ref, card_ref, w_ref, b_ref, out_ref):
    i = pl.program_id(0)

    @pl.when(i == 0)
    def _init():
        out_ref[...] = jnp.zeros_like(out_ref)

    pre = jnp.dot(card_ref[...], w_ref[...],
                  preferred_element_type=jnp.float32) + b_ref[...]
    transformed = pre * jax.nn.sigmoid(pre)
    row = jax.lax.broadcasted_iota(jnp.int32, (_BN, _M), 0) + i * _BN
    transformed = jnp.where(row < _N, transformed, 0.0)
    out_ref[...] += jax.lax.dot_general(
        meta_ref[...].astype(jnp.bfloat16), transformed.astype(jnp.bfloat16),
        (((0,), (0,)), ((), ())),
        preferred_element_type=jnp.float32)


def _pool_body(pools_ref, path_ref, out_ref):
    out_ref[...] = jnp.dot(pools_ref[...], path_ref[...],
                           preferred_element_type=jnp.float32)


def kernel(batch_pools, metapath, card_embeddings, W, b_dense):
    b2 = b_dense.reshape(1, _M)
    path = pl.pallas_call(
        _path_body,
        grid=(_G,),
        in_specs=[
            pl.BlockSpec((_BN, _P), lambda i: (i, 0)),
            pl.BlockSpec((_BN, _D), lambda i: (i, 0)),
            pl.BlockSpec((_D, _M), lambda i: (0, 0)),
            pl.BlockSpec((1, _M), lambda i: (0, 0)),
        ],
        out_specs=pl.BlockSpec((_P, _M), lambda i: (0, 0)),
        out_shape=jax.ShapeDtypeStruct((_P, _M), jnp.float32),
    )(metapath, card_embeddings, W, b2)
    return pl.pallas_call(
        _pool_body,
        in_specs=[
            pl.BlockSpec(memory_space=pltpu.VMEM),
            pl.BlockSpec(memory_space=pltpu.VMEM),
        ],
        out_specs=pl.BlockSpec(memory_space=pltpu.VMEM),
        out_shape=jax.ShapeDtypeStruct((_B, _M), jnp.float32),
    )(batch_pools, path)
